# baseline (speedup 1.0000x reference)
import dataclasses
import functools

import jax
import jax.numpy as jnp
from jax import lax
from jax.experimental import pallas as pl
from jax.experimental.pallas import tpu as pltpu
from jax.experimental.pallas import tpu_sc as plsc

_B, _S, _D = 2, 2048, 1024
_H = 2048
_MH = 64
_E = 8
_V = 32000
_N = _B * _S
_H2 = _H // 2

_BN = 256
_G = _N // _BN

_NC, _NS, _LANES = 2, 16, 16
_NW = _NC * _NS
_PER_W = _N // _NW


def _gains_sc_kernel(table_hbm, ids_hbm, out_hbm, idx_v, vals_v, sem):
    wid = lax.axis_index("s") * _NC + lax.axis_index("c")
    base = wid * _PER_W
    pltpu.sync_copy(ids_hbm.at[pl.ds(base, _PER_W)], idx_v)
    pltpu.async_copy(table_hbm.at[idx_v], vals_v, sem).wait()

    @pl.loop(0, _PER_W, step=_LANES)
    def _(i):
        v = vals_v[pl.ds(i, _LANES)]
        vals_v[pl.ds(i, _LANES)] = 1.0 / (1.0 + jnp.exp(-v)) + 0.5

    pltpu.sync_copy(vals_v, out_hbm.at[pl.ds(base, _PER_W)])


def _gains_sc(prosody_table, ids_flat):
    mesh = plsc.VectorSubcoreMesh(core_axis_name="c", subcore_axis_name="s")
    cp = pltpu.CompilerParams()
    if "needs_layout_passes" in pltpu.CompilerParams.__dataclass_fields__:
        cp = dataclasses.replace(cp, needs_layout_passes=False)
    k = pl.kernel(
        _gains_sc_kernel,
        out_type=jax.ShapeDtypeStruct((_N,), jnp.float32),
        mesh=mesh,
        scratch_types=[
            pltpu.VMEM((_PER_W,), jnp.int32),
            pltpu.VMEM((_PER_W,), jnp.float32),
            pltpu.SemaphoreType.DMA,
        ],
        compiler_params=cp,
    )
    return k(prosody_table, ids_flat)


def _dot0(w, act):
    return lax.dot_general(w, act, (((0,), (0,)), ((), ())),
                           preferred_element_type=jnp.float32)


def _tc_body(x_ref, g1_ref, gb_ref, gc_ref, gt_ref,
             wenc_ref, bcols_ref, ws2c_ref, wr1_ref, wr2_ref,
             we1_ref, be1_ref, we2_ref, be2_ref,
             wc2s_ref, bc2s_ref, wdec_ref, bdec_ref,
             lng_ref, lnb_ref, out_ref,
             cont_scr, eo_scr, wenc_bf, wdec_bf, ws2c_bf, we1e_scr, we2_bf,
             wc2sx_scr, be2p_scr, benc2_b, cdec_scr):
    f32 = jnp.float32
    bf16 = jnp.bfloat16
    i = pl.program_id(0)
    par = lax.rem(i, 2)

    @pl.when(i == 0)
    def _prep():
        wenc_bf[...] = wenc_ref[...].astype(bf16)
        wdec_bf[...] = wdec_ref[...].astype(bf16)
        ws2c_bf[...] = (ws2c_ref[...] * 0.5).astype(bf16)
        we2_bf[...] = we2_ref[...].astype(bf16)
        we1e_scr[:, :_MH] = (we1_ref[...] + we1_ref[...]).astype(bf16)
        we1e_scr[:, _MH:] = jnp.zeros((_E, 8, _H2), bf16)
        we1e_scr[:, _MH:_MH + 1] = (
            (be1_ref[...] + be1_ref[...]).astype(bf16))
        wc2sx_scr[:_MH] = (wc2s_ref[...] * 0.5).astype(bf16)
        wc2sx_scr[_MH:] = jnp.zeros((8, _H), bf16)
        wc2sx_scr[_MH:_MH + 1] = (bc2s_ref[...] * 0.5).astype(bf16)
        be2p_scr[...] = be2_ref[...] + 0.5 * jnp.sum(we2_ref[...], axis=1)
        benc2_b[...] = jnp.broadcast_to(bcols_ref[:_H], (_H, _BN))
        cdec_scr[...] = jnp.sum(wdec_ref[...], axis=0, keepdims=True)

    @pl.when(i > 1)
    def _stage_c():
        g = gc_ref[...]
        eo_ext = eo_scr[1 - par].astype(bf16)
        t_r = jnp.tanh(_dot0(wc2sx_scr[...], eo_ext))
        avg = (t_r * g).astype(bf16)

        bd = bdec_ref[...]
        z = (lax.dot_general(avg, wdec_bf[...], (((0,), (0,)), ((), ())),
                             preferred_element_type=f32)
             + (bd + bd) + gt_ref[...] * cdec_scr[...])
        t_d = jnp.tanh(z)

        mu = jnp.mean(t_d, axis=1, keepdims=True)
        var = jnp.mean((t_d - mu) ** 2, axis=1, keepdims=True)
        out_ref[...] = (((t_d - mu) * lax.rsqrt(var + 4e-5)) * lng_ref[...]
                        + lnb_ref[...])

    @pl.when((i > 0) & (i <= _G))
    def _stage_b():
        g = gb_ref[...]
        cont = cont_scr[1 - par]

        h = jnp.tanh(_dot0(wr1_ref[...], cont)
                     + bcols_ref[_H + _MH:_H + 2 * _MH])
        logits = (_dot0(wr2_ref[...], h)
                  + bcols_ref[_H + 2 * _MH:_H + 2 * _MH + _E]) * g
        m = jnp.max(logits, axis=0, keepdims=True)
        p = jnp.exp(logits - m)
        p = p / jnp.sum(p, axis=0, keepdims=True)

        eidx = lax.broadcasted_iota(jnp.int32, (_E, _BN), 0)
        m1 = jnp.max(p, axis=0, keepdims=True)
        i1 = jnp.min(jnp.where(p >= m1, eidx, _E), axis=0, keepdims=True)
        oh1 = eidx == i1
        pm = jnp.where(oh1, -1.0, p)
        m2 = jnp.max(pm, axis=0, keepdims=True)
        i2 = jnp.min(jnp.where(pm >= m2, eidx, _E), axis=0, keepdims=True)
        oh2 = eidx == i2
        denom = m1 + m2 + 1e-9
        gate = (jnp.where(oh1, m1, 0.0) + jnp.where(oh2, m2, 0.0)) / denom

        ones8 = jnp.ones((8, _BN), f32)
        cont_ext = jnp.concatenate(
            [cont, jnp.ones((1, _BN), f32), jnp.zeros((7, _BN), f32)],
            0).astype(bf16)
        ghalf = gate * 0.5
        eo = _dot0(be2p_scr[...], gate)
        for e in range(_E):
            h1_e = jnp.tanh(_dot0(we1e_scr[e], cont_ext))
            blk = (h1_e * ghalf[e:e + 1]).astype(bf16)
            eo = eo + _dot0(we2_bf[e], blk)
        eo_scr[par, :_MH] = eo
        eo_scr[par, _MH:] = ones8

    @pl.when(i < _G)
    def _stage_a():
        g = g1_ref[...]
        g2 = g + g
        x = x_ref[...].astype(bf16)
        z1 = lax.dot_general(wenc_bf[...], x, (((0,), (1,)), ((), ())),
                             preferred_element_type=f32)
        t_a = jnp.tanh(z1 * g2 + benc2_b[...])
        cont_scr[par] = (_dot0(ws2c_bf[...], t_a.astype(bf16))
                         + bcols_ref[_H:_H + _MH])


def _full(shape):
    nd = len(shape)
    return pl.BlockSpec(shape, lambda i, _nd=nd: (0,) * _nd)


def _tc_call(x, gains_row, *weights):
    def _clip(v):
        return jnp.clip(v, 0, _G - 1)

    in_specs = [
        pl.BlockSpec((_BN, _D), lambda i: (_clip(i), 0)),
        pl.BlockSpec((1, _BN), lambda i: (0, _clip(i))),
        pl.BlockSpec((1, _BN), lambda i: (0, _clip(i - 1))),
        pl.BlockSpec((1, _BN), lambda i: (0, _clip(i - 2))),
        pl.BlockSpec((_BN, 1), lambda i: (_clip(i - 2), 0)),
    ] + [_full(w.shape) for w in weights]
    f32 = jnp.float32
    return pl.pallas_call(
        _tc_body,
        grid=(_G + 2,),
        in_specs=in_specs,
        out_specs=pl.BlockSpec((_BN, _D), lambda i: (_clip(i - 2), 0)),
        out_shape=jax.ShapeDtypeStruct((_N, _D), f32),
        scratch_shapes=[
            pltpu.VMEM((2, _MH, _BN), f32),
            pltpu.VMEM((2, 72, _BN), f32),
            pltpu.VMEM((_D, _H), jnp.bfloat16),
            pltpu.VMEM((_H, _D), jnp.bfloat16),
            pltpu.VMEM((_H, _MH), jnp.bfloat16),
            pltpu.VMEM((_E, 72, _H2), jnp.bfloat16),
            pltpu.VMEM((_E, _H2, _MH), jnp.bfloat16),
            pltpu.VMEM((72, _H), jnp.bfloat16),
            pltpu.VMEM((_E, _MH), f32),
            pltpu.VMEM((_H, _BN), f32),
            pltpu.VMEM((1, _D), f32),
        ],
        compiler_params=pltpu.CompilerParams(
            dimension_semantics=("arbitrary",)),
    )(x, gains_row, gains_row, gains_row, gains_row.reshape(_N, 1), *weights)


def kernel(inputs_embeds, input_ids, prosody_table, W_enc, b_enc, W_s2c, b_s2c,
           W_r1, b_r1, W_r2, b_r2, W_e1, b_e1, W_e2, b_e2,
           W_c2s, b_c2s, W_dec, b_dec, ln_g, ln_b):
    gains = _gains_sc(prosody_table, input_ids.reshape(_N))

    bcols = jnp.concatenate(
        [2.0 * b_enc, b_s2c + 0.5 * W_s2c.sum(0), b_r1, b_r2]
    ).reshape(_H + 2 * _MH + _E, 1)

    weights = (
        W_enc,
        bcols,
        W_s2c,
        W_r1, W_r2,
        W_e1,
        b_e1.reshape(_E, 1, _H2),
        W_e2,
        b_e2,
        W_c2s,
        b_c2s.reshape(1, _H),
        W_dec,
        b_dec.reshape(1, _D),
        ln_g.reshape(1, _D),
        ln_b.reshape(1, _D),
    )
    out = _tc_call(inputs_embeds.reshape(_N, _D), gains.reshape(1, _N),
                   *weights)
    return out.reshape(_B, _S, _D)


# speedup vs baseline: 1.0047x; 1.0047x over previous
import dataclasses
import functools

import jax
import jax.numpy as jnp
from jax import lax
from jax.experimental import pallas as pl
from jax.experimental.pallas import tpu as pltpu
from jax.experimental.pallas import tpu_sc as plsc

_B, _S, _D = 2, 2048, 1024
_H = 2048
_MH = 64
_E = 8
_V = 32000
_N = _B * _S
_H2 = _H // 2

_BN = 256
_G = _N // _BN

_NC, _NS, _LANES = 2, 16, 16
_NW = _NC * _NS
_PER_W = _N // _NW


def _gains_sc_kernel(table_hbm, ids_hbm, out_hbm, idx_v, vals_v, sem):
    wid = lax.axis_index("s") * _NC + lax.axis_index("c")
    base = wid * _PER_W
    pltpu.sync_copy(ids_hbm.at[pl.ds(base, _PER_W)], idx_v)
    pltpu.async_copy(table_hbm.at[idx_v], vals_v, sem).wait()

    @pl.loop(0, _PER_W, step=_LANES)
    def _(i):
        v = vals_v[pl.ds(i, _LANES)]
        vals_v[pl.ds(i, _LANES)] = 1.0 / (1.0 + jnp.exp(-v)) + 0.5

    pltpu.sync_copy(vals_v, out_hbm.at[pl.ds(base, _PER_W)])


def _gains_sc(prosody_table, ids_flat):
    mesh = plsc.VectorSubcoreMesh(core_axis_name="c", subcore_axis_name="s")
    cp = pltpu.CompilerParams()
    if "needs_layout_passes" in pltpu.CompilerParams.__dataclass_fields__:
        cp = dataclasses.replace(cp, needs_layout_passes=False)
    k = pl.kernel(
        _gains_sc_kernel,
        out_type=jax.ShapeDtypeStruct((_N,), jnp.float32),
        mesh=mesh,
        scratch_types=[
            pltpu.VMEM((_PER_W,), jnp.int32),
            pltpu.VMEM((_PER_W,), jnp.float32),
            pltpu.SemaphoreType.DMA,
        ],
        compiler_params=cp,
    )
    return k(prosody_table, ids_flat)


def _dot0(w, act):
    return lax.dot_general(w, act, (((0,), (0,)), ((), ())),
                           preferred_element_type=jnp.float32)


def _tc_body(x_ref, g1_ref, gb_ref, gc_ref, gt_ref,
             wenc_ref, bcols_ref, ws2c_ref, wr1_ref, wr2_ref,
             we1_ref, be1_ref, we2_ref, be2_ref,
             wc2s_ref, bc2s_ref, wdec_ref, bdec_ref,
             lng_ref, lnb_ref, out_ref,
             cont_scr, eo_scr, wenc_bf, wdec_bf, ws2c_bf, we1e_scr, we2_bf,
             wc2sx_scr, be2p_scr, benc2_b, cdec_scr):
    f32 = jnp.float32
    bf16 = jnp.bfloat16
    i = pl.program_id(0)
    par = lax.rem(i, 2)

    @pl.when(i == 0)
    def _prep0():
        wenc_bf[...] = wenc_ref[...].astype(bf16)
        ws2c_bf[...] = (ws2c_ref[...] * 0.5).astype(bf16)
        we2_bf[...] = we2_ref[...].astype(bf16)
        we1e_scr[:, :_MH] = (we1_ref[...] + we1_ref[...]).astype(bf16)
        we1e_scr[:, _MH:] = jnp.zeros((_E, 8, _H2), bf16)
        we1e_scr[:, _MH:_MH + 1] = (
            (be1_ref[...] + be1_ref[...]).astype(bf16))
        be2p_scr[...] = be2_ref[...] + 0.5 * jnp.sum(we2_ref[...], axis=1)
        benc2_b[...] = jnp.broadcast_to(bcols_ref[:_H], (_H, _BN))

    @pl.when(i == 1)
    def _prep1():
        wdec_bf[...] = wdec_ref[...].astype(bf16)
        wc2sx_scr[:_MH] = (wc2s_ref[...] * 0.5).astype(bf16)
        wc2sx_scr[_MH:] = jnp.zeros((8, _H), bf16)
        wc2sx_scr[_MH:_MH + 1] = (bc2s_ref[...] * 0.5).astype(bf16)
        cdec_scr[...] = jnp.sum(wdec_ref[...], axis=0, keepdims=True)

    @pl.when(i > 1)
    def _stage_c():
        g = gc_ref[...]
        eo_ext = eo_scr[1 - par].astype(bf16)
        t_r = jnp.tanh(_dot0(wc2sx_scr[...], eo_ext))
        avg = (t_r * g).astype(bf16)

        bd = bdec_ref[...]
        z = (lax.dot_general(avg, wdec_bf[...], (((0,), (0,)), ((), ())),
                             preferred_element_type=f32)
             + (bd + bd) + gt_ref[...] * cdec_scr[...])
        t_d = jnp.tanh(z)

        mu = jnp.mean(t_d, axis=1, keepdims=True)
        var = jnp.mean((t_d - mu) ** 2, axis=1, keepdims=True)
        out_ref[...] = (((t_d - mu) * lax.rsqrt(var + 4e-5)) * lng_ref[...]
                        + lnb_ref[...])

    @pl.when((i > 0) & (i <= _G))
    def _stage_b():
        g = gb_ref[...]
        cont = cont_scr[1 - par]

        h = jnp.tanh(_dot0(wr1_ref[...], cont)
                     + bcols_ref[_H + _MH:_H + 2 * _MH])
        logits = (_dot0(wr2_ref[...], h)
                  + bcols_ref[_H + 2 * _MH:_H + 2 * _MH + _E]) * g
        m = jnp.max(logits, axis=0, keepdims=True)
        p = jnp.exp(logits - m)
        p = p / jnp.sum(p, axis=0, keepdims=True)

        eidx = lax.broadcasted_iota(jnp.int32, (_E, _BN), 0)
        m1 = jnp.max(p, axis=0, keepdims=True)
        i1 = jnp.min(jnp.where(p >= m1, eidx, _E), axis=0, keepdims=True)
        oh1 = eidx == i1
        pm = jnp.where(oh1, -1.0, p)
        m2 = jnp.max(pm, axis=0, keepdims=True)
        i2 = jnp.min(jnp.where(pm >= m2, eidx, _E), axis=0, keepdims=True)
        oh2 = eidx == i2
        denom = m1 + m2 + 1e-9
        gate = (jnp.where(oh1, m1, 0.0) + jnp.where(oh2, m2, 0.0)) / denom

        ones8 = jnp.ones((8, _BN), f32)
        cont_ext = jnp.concatenate(
            [cont, jnp.ones((1, _BN), f32), jnp.zeros((7, _BN), f32)],
            0).astype(bf16)
        ghalf = gate * 0.5
        eo = _dot0(be2p_scr[...], gate)
        for e in range(_E):
            h1_e = jnp.tanh(_dot0(we1e_scr[e], cont_ext))
            blk = (h1_e * ghalf[e:e + 1]).astype(bf16)
            eo = eo + _dot0(we2_bf[e], blk)
        eo_scr[par, :_MH] = eo
        eo_scr[par, _MH:] = ones8

    @pl.when(i < _G)
    def _stage_a():
        g = g1_ref[...]
        g2 = g + g
        x = x_ref[...].astype(bf16)
        z1 = lax.dot_general(wenc_bf[...], x, (((0,), (1,)), ((), ())),
                             preferred_element_type=f32)
        t_a = jnp.tanh(z1 * g2 + benc2_b[...])
        cont_scr[par] = (_dot0(ws2c_bf[...], t_a.astype(bf16))
                         + bcols_ref[_H:_H + _MH])


def _full(shape):
    nd = len(shape)
    return pl.BlockSpec(shape, lambda i, _nd=nd: (0,) * _nd)


def _tc_call(x, gains_row, *weights):
    def _clip(v):
        return jnp.clip(v, 0, _G - 1)

    in_specs = [
        pl.BlockSpec((_BN, _D), lambda i: (_clip(i), 0)),
        pl.BlockSpec((1, _BN), lambda i: (0, _clip(i))),
        pl.BlockSpec((1, _BN), lambda i: (0, _clip(i - 1))),
        pl.BlockSpec((1, _BN), lambda i: (0, _clip(i - 2))),
        pl.BlockSpec((_BN, 1), lambda i: (_clip(i - 2), 0)),
    ] + [_full(w.shape) for w in weights]
    f32 = jnp.float32
    return pl.pallas_call(
        _tc_body,
        grid=(_G + 2,),
        in_specs=in_specs,
        out_specs=pl.BlockSpec((_BN, _D), lambda i: (_clip(i - 2), 0)),
        out_shape=jax.ShapeDtypeStruct((_N, _D), f32),
        scratch_shapes=[
            pltpu.VMEM((2, _MH, _BN), f32),
            pltpu.VMEM((2, 72, _BN), f32),
            pltpu.VMEM((_D, _H), jnp.bfloat16),
            pltpu.VMEM((_H, _D), jnp.bfloat16),
            pltpu.VMEM((_H, _MH), jnp.bfloat16),
            pltpu.VMEM((_E, 72, _H2), jnp.bfloat16),
            pltpu.VMEM((_E, _H2, _MH), jnp.bfloat16),
            pltpu.VMEM((72, _H), jnp.bfloat16),
            pltpu.VMEM((_E, _MH), f32),
            pltpu.VMEM((_H, _BN), f32),
            pltpu.VMEM((1, _D), f32),
        ],
        compiler_params=pltpu.CompilerParams(
            dimension_semantics=("arbitrary",)),
    )(x, gains_row, gains_row, gains_row, gains_row.reshape(_N, 1), *weights)


def kernel(inputs_embeds, input_ids, prosody_table, W_enc, b_enc, W_s2c, b_s2c,
           W_r1, b_r1, W_r2, b_r2, W_e1, b_e1, W_e2, b_e2,
           W_c2s, b_c2s, W_dec, b_dec, ln_g, ln_b):
    gains = _gains_sc(prosody_table, input_ids.reshape(_N))

    bcols = jnp.concatenate(
        [2.0 * b_enc, b_s2c + 0.5 * W_s2c.sum(0), b_r1, b_r2]
    ).reshape(_H + 2 * _MH + _E, 1)

    weights = (
        W_enc,
        bcols,
        W_s2c,
        W_r1, W_r2,
        W_e1,
        b_e1.reshape(_E, 1, _H2),
        W_e2,
        b_e2,
        W_c2s,
        b_c2s.reshape(1, _H),
        W_dec,
        b_dec.reshape(1, _D),
        ln_g.reshape(1, _D),
        ln_b.reshape(1, _D),
    )
    out = _tc_call(inputs_embeds.reshape(_N, _D), gains.reshape(1, _N),
                   *weights)
    return out.reshape(_B, _S, _D)


# speedup vs baseline: 1.1107x; 1.1055x over previous
import dataclasses
import functools

import jax
import jax.numpy as jnp
from jax import lax
from jax.experimental import pallas as pl
from jax.experimental.pallas import tpu as pltpu
from jax.experimental.pallas import tpu_sc as plsc

_B, _S, _D = 2, 2048, 1024
_H = 2048
_MH = 64
_E = 8
_V = 32000
_N = _B * _S
_H2 = _H // 2

_BN = 512
_G = _N // _BN

_NC, _NS, _LANES = 2, 16, 16
_NW = _NC * _NS
_PER_W = _N // _NW


def _gains_sc_kernel(table_hbm, ids_hbm, out_hbm, idx_v, vals_v, sem):
    wid = lax.axis_index("s") * _NC + lax.axis_index("c")
    base = wid * _PER_W
    pltpu.sync_copy(ids_hbm.at[pl.ds(base, _PER_W)], idx_v)
    pltpu.async_copy(table_hbm.at[idx_v], vals_v, sem).wait()

    @pl.loop(0, _PER_W, step=_LANES)
    def _(i):
        v = vals_v[pl.ds(i, _LANES)]
        vals_v[pl.ds(i, _LANES)] = 1.0 / (1.0 + jnp.exp(-v)) + 0.5

    pltpu.sync_copy(vals_v, out_hbm.at[pl.ds(base, _PER_W)])


def _gains_sc(prosody_table, ids_flat):
    mesh = plsc.VectorSubcoreMesh(core_axis_name="c", subcore_axis_name="s")
    cp = pltpu.CompilerParams()
    if "needs_layout_passes" in pltpu.CompilerParams.__dataclass_fields__:
        cp = dataclasses.replace(cp, needs_layout_passes=False)
    k = pl.kernel(
        _gains_sc_kernel,
        out_type=jax.ShapeDtypeStruct((_N,), jnp.float32),
        mesh=mesh,
        scratch_types=[
            pltpu.VMEM((_PER_W,), jnp.int32),
            pltpu.VMEM((_PER_W,), jnp.float32),
            pltpu.SemaphoreType.DMA,
        ],
        compiler_params=cp,
    )
    return k(prosody_table, ids_flat)


def _dot0(w, act):
    return lax.dot_general(w, act, (((0,), (0,)), ((), ())),
                           preferred_element_type=jnp.float32)


def _tc_body(x_ref, g1_ref, gb_ref, gc_ref, gt_ref,
             wenc_ref, bcols_ref, ws2c_ref, wr1_ref, wr2_ref,
             we1_ref, be1_ref, we2_ref, be2_ref,
             wc2s_ref, bc2s_ref, wdec_ref, bdec_ref,
             lng_ref, lnb_ref, out_ref,
             cont_scr, eo_scr, wenc_bf, wdec_bf, ws2c_bf, we1e_scr, we2_bf,
             wc2sx_scr, be2p_scr, benc2_b, cdec_scr):
    f32 = jnp.float32
    bf16 = jnp.bfloat16
    i = pl.program_id(0)
    par = lax.rem(i, 2)

    @pl.when(i == 0)
    def _prep0():
        wenc_bf[...] = wenc_ref[...].astype(bf16)
        ws2c_bf[...] = (ws2c_ref[...] * 0.5).astype(bf16)
        we2_bf[...] = we2_ref[...].astype(bf16)
        we1e_scr[:, :_MH] = (we1_ref[...] + we1_ref[...]).astype(bf16)
        we1e_scr[:, _MH:] = jnp.zeros((_E, 8, _H2), bf16)
        we1e_scr[:, _MH:_MH + 1] = (
            (be1_ref[...] + be1_ref[...]).astype(bf16))
        be2p_scr[...] = be2_ref[...] + 0.5 * jnp.sum(we2_ref[...], axis=1)
        benc2_b[...] = jnp.broadcast_to(bcols_ref[:_H], (_H, _BN))

    @pl.when(i == 1)
    def _prep1():
        wdec_bf[...] = wdec_ref[...].astype(bf16)
        wc2sx_scr[:_MH] = (wc2s_ref[...] * 0.5).astype(bf16)
        wc2sx_scr[_MH:] = jnp.zeros((8, _H), bf16)
        wc2sx_scr[_MH:_MH + 1] = (bc2s_ref[...] * 0.5).astype(bf16)
        cdec_scr[...] = jnp.sum(wdec_ref[...], axis=0, keepdims=True)

    @pl.when(i > 1)
    def _stage_c():
        g = gc_ref[...]
        eo_ext = eo_scr[1 - par].astype(bf16)
        t_r = jnp.tanh(_dot0(wc2sx_scr[...], eo_ext))
        avg = (t_r * g).astype(bf16)

        bd = bdec_ref[...]
        z = (lax.dot_general(avg, wdec_bf[...], (((0,), (0,)), ((), ())),
                             preferred_element_type=f32)
             + (bd + bd) + gt_ref[...] * cdec_scr[...])
        t_d = jnp.tanh(z)

        mu = jnp.mean(t_d, axis=1, keepdims=True)
        var = jnp.mean((t_d - mu) ** 2, axis=1, keepdims=True)
        out_ref[...] = (((t_d - mu) * lax.rsqrt(var + 4e-5)) * lng_ref[...]
                        + lnb_ref[...])

    @pl.when((i > 0) & (i <= _G))
    def _stage_b():
        g = gb_ref[...]
        cont = cont_scr[1 - par]

        h = jnp.tanh(_dot0(wr1_ref[...], cont)
                     + bcols_ref[_H + _MH:_H + 2 * _MH])
        logits = (_dot0(wr2_ref[...], h)
                  + bcols_ref[_H + 2 * _MH:_H + 2 * _MH + _E]) * g
        m = jnp.max(logits, axis=0, keepdims=True)
        p = jnp.exp(logits - m)
        p = p / jnp.sum(p, axis=0, keepdims=True)

        eidx = lax.broadcasted_iota(jnp.int32, (_E, _BN), 0)
        m1 = jnp.max(p, axis=0, keepdims=True)
        i1 = jnp.min(jnp.where(p >= m1, eidx, _E), axis=0, keepdims=True)
        oh1 = eidx == i1
        pm = jnp.where(oh1, -1.0, p)
        m2 = jnp.max(pm, axis=0, keepdims=True)
        i2 = jnp.min(jnp.where(pm >= m2, eidx, _E), axis=0, keepdims=True)
        oh2 = eidx == i2
        denom = m1 + m2 + 1e-9
        gate = (jnp.where(oh1, m1, 0.0) + jnp.where(oh2, m2, 0.0)) / denom

        ones8 = jnp.ones((8, _BN), f32)
        cont_ext = jnp.concatenate(
            [cont, jnp.ones((1, _BN), f32), jnp.zeros((7, _BN), f32)],
            0).astype(bf16)
        ghalf = gate * 0.5
        eo = _dot0(be2p_scr[...], gate)
        for e in range(_E):
            h1_e = jnp.tanh(_dot0(we1e_scr[e], cont_ext))
            blk = (h1_e * ghalf[e:e + 1]).astype(bf16)
            eo = eo + _dot0(we2_bf[e], blk)
        eo_scr[par, :_MH] = eo
        eo_scr[par, _MH:] = ones8

    @pl.when(i < _G)
    def _stage_a():
        g = g1_ref[...]
        g2 = g + g
        x = x_ref[...].astype(bf16)
        z1 = lax.dot_general(wenc_bf[...], x, (((0,), (1,)), ((), ())),
                             preferred_element_type=f32)
        t_a = jnp.tanh(z1 * g2 + benc2_b[...])
        cont_scr[par] = (_dot0(ws2c_bf[...], t_a.astype(bf16))
                         + bcols_ref[_H:_H + _MH])


def _full(shape):
    nd = len(shape)
    return pl.BlockSpec(shape, lambda i, _nd=nd: (0,) * _nd)


def _tc_call(x, gains_row, *weights):
    def _clip(v):
        return jnp.clip(v, 0, _G - 1)

    in_specs = [
        pl.BlockSpec((_BN, _D), lambda i: (_clip(i), 0)),
        pl.BlockSpec((1, _BN), lambda i: (0, _clip(i))),
        pl.BlockSpec((1, _BN), lambda i: (0, _clip(i - 1))),
        pl.BlockSpec((1, _BN), lambda i: (0, _clip(i - 2))),
        pl.BlockSpec((_BN, 1), lambda i: (_clip(i - 2), 0)),
    ] + [_full(w.shape) for w in weights]
    f32 = jnp.float32
    return pl.pallas_call(
        _tc_body,
        grid=(_G + 2,),
        in_specs=in_specs,
        out_specs=pl.BlockSpec((_BN, _D), lambda i: (_clip(i - 2), 0)),
        out_shape=jax.ShapeDtypeStruct((_N, _D), f32),
        scratch_shapes=[
            pltpu.VMEM((2, _MH, _BN), f32),
            pltpu.VMEM((2, 72, _BN), f32),
            pltpu.VMEM((_D, _H), jnp.bfloat16),
            pltpu.VMEM((_H, _D), jnp.bfloat16),
            pltpu.VMEM((_H, _MH), jnp.bfloat16),
            pltpu.VMEM((_E, 72, _H2), jnp.bfloat16),
            pltpu.VMEM((_E, _H2, _MH), jnp.bfloat16),
            pltpu.VMEM((72, _H), jnp.bfloat16),
            pltpu.VMEM((_E, _MH), f32),
            pltpu.VMEM((_H, _BN), f32),
            pltpu.VMEM((1, _D), f32),
        ],
        compiler_params=pltpu.CompilerParams(
            dimension_semantics=("arbitrary",)),
    )(x, gains_row, gains_row, gains_row, gains_row.reshape(_N, 1), *weights)


def kernel(inputs_embeds, input_ids, prosody_table, W_enc, b_enc, W_s2c, b_s2c,
           W_r1, b_r1, W_r2, b_r2, W_e1, b_e1, W_e2, b_e2,
           W_c2s, b_c2s, W_dec, b_dec, ln_g, ln_b):
    gains = _gains_sc(prosody_table, input_ids.reshape(_N))

    bcols = jnp.concatenate(
        [2.0 * b_enc, b_s2c + 0.5 * W_s2c.sum(0), b_r1, b_r2]
    ).reshape(_H + 2 * _MH + _E, 1)

    weights = (
        W_enc,
        bcols,
        W_s2c,
        W_r1, W_r2,
        W_e1,
        b_e1.reshape(_E, 1, _H2),
        W_e2,
        b_e2,
        W_c2s,
        b_c2s.reshape(1, _H),
        W_dec,
        b_dec.reshape(1, _D),
        ln_g.reshape(1, _D),
        ln_b.reshape(1, _D),
    )
    out = _tc_call(inputs_embeds.reshape(_N, _D), gains.reshape(1, _N),
                   *weights)
    return out.reshape(_B, _S, _D)


# speedup vs baseline: 1.1156x; 1.0044x over previous
import dataclasses
import functools

import jax
import jax.numpy as jnp
from jax import lax
from jax.experimental import pallas as pl
from jax.experimental.pallas import tpu as pltpu
from jax.experimental.pallas import tpu_sc as plsc

_B, _S, _D = 2, 2048, 1024
_H = 2048
_MH = 64
_E = 8
_V = 32000
_N = _B * _S
_H2 = _H // 2

_BN = 512
_G = _N // _BN

_NC, _NS, _LANES = 2, 16, 16
_NW = _NC * _NS
_PER_W = _N // _NW


def _gains_sc_kernel(table_hbm, ids_hbm, out_hbm, idx_v, vals_v, sem):
    wid = lax.axis_index("s") * _NC + lax.axis_index("c")
    base = wid * _PER_W
    pltpu.sync_copy(ids_hbm.at[pl.ds(base, _PER_W)], idx_v)
    pltpu.async_copy(table_hbm.at[idx_v], vals_v, sem).wait()

    @pl.loop(0, _PER_W, step=_LANES)
    def _(i):
        v = vals_v[pl.ds(i, _LANES)]
        vals_v[pl.ds(i, _LANES)] = 1.0 / (1.0 + jnp.exp(-v)) + 0.5

    pltpu.sync_copy(vals_v, out_hbm.at[pl.ds(base, _PER_W)])


def _gains_sc(prosody_table, ids_flat):
    mesh = plsc.VectorSubcoreMesh(core_axis_name="c", subcore_axis_name="s")
    cp = pltpu.CompilerParams()
    if "needs_layout_passes" in pltpu.CompilerParams.__dataclass_fields__:
        cp = dataclasses.replace(cp, needs_layout_passes=False)
    k = pl.kernel(
        _gains_sc_kernel,
        out_type=jax.ShapeDtypeStruct((_N,), jnp.float32),
        mesh=mesh,
        scratch_types=[
            pltpu.VMEM((_PER_W,), jnp.int32),
            pltpu.VMEM((_PER_W,), jnp.float32),
            pltpu.SemaphoreType.DMA,
        ],
        compiler_params=cp,
    )
    return k(prosody_table, ids_flat)


def _dot0(w, act):
    return lax.dot_general(w, act, (((0,), (0,)), ((), ())),
                           preferred_element_type=jnp.float32)


def _tc_body(x_ref, g1_ref, gb_ref, gc_ref,
             wenc_ref, bcols_ref, ws2c_ref, wr1_ref, wr2_ref,
             we1_ref, be1_ref, we2_ref, be2_ref,
             wc2s_ref, bc2s_ref, wdec_ref, bdec_ref,
             lng_ref, lnb_ref, out_ref,
             cont_scr, eo_scr, wenc_bf, wdec_bf, ws2c_bf, we1e_scr, we2_bf,
             wc2sx_scr, be2p_scr, benc2_b, cdec_scr):
    f32 = jnp.float32
    bf16 = jnp.bfloat16
    i = pl.program_id(0)
    par = lax.rem(i, 2)

    @pl.when(i == 0)
    def _prep0():
        wenc_bf[...] = wenc_ref[...].astype(bf16)
        ws2c_bf[...] = (ws2c_ref[...] * 0.5).astype(bf16)
        we2_bf[...] = we2_ref[...].astype(bf16)
        we1e_scr[:, :_MH] = (we1_ref[...] + we1_ref[...]).astype(bf16)
        we1e_scr[:, _MH:] = jnp.zeros((_E, 8, _H2), bf16)
        we1e_scr[:, _MH:_MH + 1] = (
            (be1_ref[...] + be1_ref[...]).astype(bf16))
        be2p_scr[...] = be2_ref[...] + 0.5 * jnp.sum(we2_ref[...], axis=1)
        benc2_b[...] = jnp.broadcast_to(bcols_ref[:_H], (_H, _BN))

    @pl.when(i == 1)
    def _prep1():
        wdec_bf[...] = wdec_ref[...].astype(bf16)
        wc2sx_scr[:_MH] = (wc2s_ref[...] * 0.5).astype(bf16)
        wc2sx_scr[_MH:] = jnp.zeros((8, _H), bf16)
        wc2sx_scr[_MH:_MH + 1] = (bc2s_ref[...] * 0.5).astype(bf16)
        cdec_scr[...] = jnp.sum(wdec_ref[...], axis=0, keepdims=True)

    @pl.when(i > 1)
    def _stage_c():
        g = gc_ref[...]
        eo_ext = eo_scr[1 - par].astype(bf16)
        t_r = jnp.tanh(_dot0(wc2sx_scr[...], eo_ext))
        avg = (t_r * g).astype(bf16)

        bd = bdec_ref[...]
        gt = jnp.transpose(g, (1, 0))
        z = (lax.dot_general(avg, wdec_bf[...], (((0,), (0,)), ((), ())),
                             preferred_element_type=f32)
             + (bd + bd) + gt * cdec_scr[...])
        t_d = jnp.tanh(z)

        mu = jnp.mean(t_d, axis=1, keepdims=True)
        var = jnp.mean((t_d - mu) ** 2, axis=1, keepdims=True)
        out_ref[...] = (((t_d - mu) * lax.rsqrt(var + 4e-5)) * lng_ref[...]
                        + lnb_ref[...])

    @pl.when((i > 0) & (i <= _G))
    def _stage_b():
        g = gb_ref[...]
        cont = cont_scr[1 - par]

        h = jnp.tanh(_dot0(wr1_ref[...], cont)
                     + bcols_ref[_H + _MH:_H + 2 * _MH])
        logits = (_dot0(wr2_ref[...], h)
                  + bcols_ref[_H + 2 * _MH:_H + 2 * _MH + _E]) * g
        m = jnp.max(logits, axis=0, keepdims=True)
        p = jnp.exp(logits - m)
        p = p / jnp.sum(p, axis=0, keepdims=True)

        eidx = lax.broadcasted_iota(jnp.int32, (_E, _BN), 0)
        m1 = jnp.max(p, axis=0, keepdims=True)
        i1 = jnp.min(jnp.where(p >= m1, eidx, _E), axis=0, keepdims=True)
        oh1 = eidx == i1
        pm = jnp.where(oh1, -1.0, p)
        m2 = jnp.max(pm, axis=0, keepdims=True)
        i2 = jnp.min(jnp.where(pm >= m2, eidx, _E), axis=0, keepdims=True)
        oh2 = eidx == i2
        denom = m1 + m2 + 1e-9
        gate = (jnp.where(oh1, m1, 0.0) + jnp.where(oh2, m2, 0.0)) / denom

        ones8 = jnp.ones((8, _BN), f32)
        cont_ext = jnp.concatenate(
            [cont, jnp.ones((1, _BN), f32), jnp.zeros((7, _BN), f32)],
            0).astype(bf16)
        ghalf = gate * 0.5
        eo = _dot0(be2p_scr[...], gate)
        for e in range(_E):
            h1_e = jnp.tanh(_dot0(we1e_scr[e], cont_ext))
            blk = (h1_e * ghalf[e:e + 1]).astype(bf16)
            eo = eo + _dot0(we2_bf[e], blk)
        eo_scr[par, :_MH] = eo
        eo_scr[par, _MH:] = ones8

    @pl.when(i < _G)
    def _stage_a():
        g = g1_ref[...]
        g2 = g + g
        x = x_ref[...].astype(bf16)
        z1 = lax.dot_general(wenc_bf[...], x, (((0,), (1,)), ((), ())),
                             preferred_element_type=f32)
        t_a = jnp.tanh(z1 * g2 + benc2_b[...])
        cont_scr[par] = (_dot0(ws2c_bf[...], t_a.astype(bf16))
                         + bcols_ref[_H:_H + _MH])


def _full(shape):
    nd = len(shape)
    return pl.BlockSpec(shape, lambda i, _nd=nd: (0,) * _nd)


def _tc_call(x, gains_row, *weights):
    def _clip(v):
        return jnp.clip(v, 0, _G - 1)

    in_specs = [
        pl.BlockSpec((_BN, _D), lambda i: (_clip(i), 0)),
        pl.BlockSpec((1, _BN), lambda i: (0, _clip(i))),
        pl.BlockSpec((1, _BN), lambda i: (0, _clip(i - 1))),
        pl.BlockSpec((1, _BN), lambda i: (0, _clip(i - 2))),
    ] + [_full(w.shape) for w in weights]
    f32 = jnp.float32
    return pl.pallas_call(
        _tc_body,
        grid=(_G + 2,),
        in_specs=in_specs,
        out_specs=pl.BlockSpec((_BN, _D), lambda i: (_clip(i - 2), 0)),
        out_shape=jax.ShapeDtypeStruct((_N, _D), f32),
        scratch_shapes=[
            pltpu.VMEM((2, _MH, _BN), f32),
            pltpu.VMEM((2, 72, _BN), f32),
            pltpu.VMEM((_D, _H), jnp.bfloat16),
            pltpu.VMEM((_H, _D), jnp.bfloat16),
            pltpu.VMEM((_H, _MH), jnp.bfloat16),
            pltpu.VMEM((_E, 72, _H2), jnp.bfloat16),
            pltpu.VMEM((_E, _H2, _MH), jnp.bfloat16),
            pltpu.VMEM((72, _H), jnp.bfloat16),
            pltpu.VMEM((_E, _MH), f32),
            pltpu.VMEM((_H, _BN), f32),
            pltpu.VMEM((1, _D), f32),
        ],
        compiler_params=pltpu.CompilerParams(
            dimension_semantics=("arbitrary",)),
    )(x, gains_row, gains_row, gains_row, *weights)


def kernel(inputs_embeds, input_ids, prosody_table, W_enc, b_enc, W_s2c, b_s2c,
           W_r1, b_r1, W_r2, b_r2, W_e1, b_e1, W_e2, b_e2,
           W_c2s, b_c2s, W_dec, b_dec, ln_g, ln_b):
    gains = _gains_sc(prosody_table, input_ids.reshape(_N))

    bcols = jnp.concatenate(
        [2.0 * b_enc, b_s2c + 0.5 * W_s2c.sum(0), b_r1, b_r2]
    ).reshape(_H + 2 * _MH + _E, 1)

    weights = (
        W_enc,
        bcols,
        W_s2c,
        W_r1, W_r2,
        W_e1,
        b_e1.reshape(_E, 1, _H2),
        W_e2,
        b_e2,
        W_c2s,
        b_c2s.reshape(1, _H),
        W_dec,
        b_dec.reshape(1, _D),
        ln_g.reshape(1, _D),
        ln_b.reshape(1, _D),
    )
    out = _tc_call(inputs_embeds.reshape(_N, _D), gains.reshape(1, _N),
                   *weights)
    return out.reshape(_B, _S, _D)
